# baseline (speedup 1.0000x reference)
import math

import jax
import jax.numpy as jnp
from jax import lax
from jax.experimental import pallas as pl
from jax.experimental.pallas import tpu as pltpu
from jax.experimental.pallas import tpu_sc as plsc

N = 10000
E = 320000
M = 2 * E
D = 20
W = 32
NC, NS = 2, 16
NW = NC * NS
CH = 80
NPAD = 10240
ROWS_PER_TILE = NPAD // NS

K2_CHUNKS, K2_G = 250, 10
K4_CHUNKS, K4_G = 125, 5
GE = K4_G * CH

BLK = 4000
PR = BLK * W // 128
EFR = E * W // 128

_SHIFT = 15.0


def _node_pre_body(x_ref, heh_ref, hz1_ref, wxp_ref, wab_ref, wqkvs_ref,
                   xp_ref, ab0_ref, q_ref, k_ref, v_ref, sk_ref):
    xp_ref[...] = jnp.dot(x_ref[...], wxp_ref[...],
                          preferred_element_type=jnp.float32)
    ab0_ref[...] = jnp.dot(heh_ref[...], wab_ref[...],
                           preferred_element_type=jnp.float32)
    qkvs = jnp.dot(hz1_ref[...], wqkvs_ref[...],
                   preferred_element_type=jnp.float32)
    zpad = jnp.zeros((qkvs.shape[0], W - D), jnp.float32)
    q_ref[...] = jnp.concatenate([qkvs[:, 0:D], zpad], axis=1)
    k_ref[...] = jnp.concatenate([qkvs[:, D:2 * D], zpad], axis=1)
    v_ref[...] = jnp.concatenate([qkvs[:, 2 * D:3 * D], zpad], axis=1)
    sk_ref[...] = qkvs[:, 3 * D:4 * D]


def _node_pre(x, heh, hz1, wxp, wab, wqkvs):
    blk = 400
    grid = N // blk
    return pl.pallas_call(
        _node_pre_body,
        grid=(grid,),
        in_specs=[
            pl.BlockSpec((blk, 128), lambda i: (i, 0)),
            pl.BlockSpec((blk, 8), lambda i: (i, 0)),
            pl.BlockSpec((blk, 24), lambda i: (i, 0)),
            pl.BlockSpec((128, 40), lambda i: (0, 0)),
            pl.BlockSpec((8, 40), lambda i: (0, 0)),
            pl.BlockSpec((24, 80), lambda i: (0, 0)),
        ],
        out_specs=[
            pl.BlockSpec((blk, 40), lambda i: (i, 0)),
            pl.BlockSpec((blk, 40), lambda i: (i, 0)),
            pl.BlockSpec((blk, W), lambda i: (i, 0)),
            pl.BlockSpec((blk, W), lambda i: (i, 0)),
            pl.BlockSpec((blk, W), lambda i: (i, 0)),
            pl.BlockSpec((blk, D), lambda i: (i, 0)),
        ],
        out_shape=[
            jax.ShapeDtypeStruct((N, 40), jnp.float32),
            jax.ShapeDtypeStruct((N, 40), jnp.float32),
            jax.ShapeDtypeStruct((N, W), jnp.float32),
            jax.ShapeDtypeStruct((N, W), jnp.float32),
            jax.ShapeDtypeStruct((N, W), jnp.float32),
            jax.ShapeDtypeStruct((N, D), jnp.float32),
        ],
    )(x, heh, hz1, wxp, wab, wqkvs)


def _seg40_body(xp_hbm, src_hbm, dst_hbm, zer_hbm, part_hbm,
                idxs_v, idxd_v, rows_v, accum_sh, sem):
    cid = lax.axis_index("c")
    sid = lax.axis_index("s")
    wid = cid * NS + sid
    pltpu.sync_copy(zer_hbm, accum_sh.at[pl.ds(sid * ROWS_PER_TILE,
                                               ROWS_PER_TILE)])
    ebase = wid * (K2_CHUNKS * CH)
    pltpu.sync_copy(src_hbm.at[pl.ds(ebase, K2_CHUNKS * CH)], idxs_v)
    pltpu.sync_copy(dst_hbm.at[pl.ds(ebase, K2_CHUNKS * CH)], idxd_v)
    plsc.subcore_barrier()

    def group(g, carry):
        g0 = pl.multiple_of(g * (K2_G * CH), 8)
        descs = []
        for j in range(K2_G):
            descs.append(pltpu.async_copy(
                xp_hbm.at[idxs_v.at[pl.ds(g0 + j * CH, CH)]],
                rows_v.at[pl.ds(j * CH, CH)], sem))
        for dsc in descs:
            dsc.wait()
        descs = []
        for j in range(K2_G):
            descs.append(pltpu.async_copy(
                rows_v.at[pl.ds(j * CH, CH)],
                accum_sh.at[idxd_v.at[pl.ds(g0 + j * CH, CH)]],
                sem, add=True))
        for dsc in descs:
            dsc.wait()
        return carry

    lax.fori_loop(0, K2_CHUNKS // K2_G, group, 0)
    plsc.subcore_barrier()
    sl = pl.ds(sid * ROWS_PER_TILE, ROWS_PER_TILE)
    pltpu.sync_copy(accum_sh.at[sl], part_hbm.at[cid].at[sl])


def _seg40(xp, src3d, dst3d, zer40):
    mesh = plsc.VectorSubcoreMesh(core_axis_name="c", subcore_axis_name="s",
                                  num_cores=NC, num_subcores=NS)
    return pl.kernel(
        _seg40_body,
        out_type=jax.ShapeDtypeStruct((NC, NPAD, 40), jnp.float32),
        mesh=mesh,
        compiler_params=pltpu.CompilerParams(use_tc_tiling_on_sc=False),
        scratch_types=[
            pltpu.VMEM((K2_CHUNKS * CH,), jnp.int32),
            pltpu.VMEM((K2_CHUNKS * CH,), jnp.int32),
            pltpu.VMEM((K2_G * CH, 40), jnp.float32),
            pltpu.VMEM_SHARED((NPAD, 40), jnp.float32),
            pltpu.SemaphoreType.DMA,
        ],
    )(xp, src3d, dst3d, zer40)


def _combine_body(part_ref, ab0_ref, a_ref, b_ref):
    p = part_ref[0] + part_ref[1] + ab0_ref[...]
    zpad = jnp.zeros((p.shape[0], W - D), jnp.float32)
    a_ref[...] = jnp.concatenate([p[:, 0:D], zpad], axis=1)
    b_ref[...] = jnp.concatenate([p[:, D:2 * D], zpad], axis=1)


def _combine(part, ab0):
    blk = 2000
    grid = N // blk
    return pl.pallas_call(
        _combine_body,
        grid=(grid,),
        in_specs=[
            pl.BlockSpec((NC, blk, 40), lambda i: (0, i, 0)),
            pl.BlockSpec((blk, 40), lambda i: (i, 0)),
        ],
        out_specs=[
            pl.BlockSpec((blk, W), lambda i: (i, 0)),
            pl.BlockSpec((blk, W), lambda i: (i, 0)),
        ],
        out_shape=[
            jax.ShapeDtypeStruct((N, W), jnp.float32),
            jax.ShapeDtypeStruct((N, W), jnp.float32),
        ],
    )(part, ab0)


def _edge_gather_body(q_hbm, k_hbm, v_hbm, a_hbm, b_hbm,
                      es_hbm, ed_hbm, sn_hbm, dn_hbm,
                      qg_hbm, kg_hbm, vg_hbm, ag_hbm, bg_hbm,
                      ixs, ixd, ixsn, ixdn, qb, kb, vb, ab, bb, sem):
    wid = lax.axis_index("c") * NS + lax.axis_index("s")
    gcnt = K4_CHUNKS // K4_G
    tbase = wid * (K4_CHUNKS * CH)
    pltpu.sync_copy(es_hbm.at[pl.ds(tbase, K4_CHUNKS * CH)], ixs)
    pltpu.sync_copy(ed_hbm.at[pl.ds(tbase, K4_CHUNKS * CH)], ixd)
    pltpu.sync_copy(sn_hbm.at[pl.ds(tbase, K4_CHUNKS * CH)], ixsn)
    pltpu.sync_copy(dn_hbm.at[pl.ds(tbase, K4_CHUNKS * CH)], ixdn)

    def group(g, carry):
        g0 = pl.multiple_of(g * GE, 8)
        descs = []
        for j in range(K4_G):
            sl = pl.ds(j * CH, CH)
            il = pl.ds(g0 + j * CH, CH)
            descs.append(pltpu.async_copy(q_hbm.at[ixd.at[il]], qb.at[sl], sem))
            descs.append(pltpu.async_copy(k_hbm.at[ixs.at[il]], kb.at[sl], sem))
            descs.append(pltpu.async_copy(v_hbm.at[ixs.at[il]], vb.at[sl], sem))
            descs.append(pltpu.async_copy(a_hbm.at[ixsn.at[il]], ab.at[sl], sem))
            descs.append(pltpu.async_copy(b_hbm.at[ixdn.at[il]], bb.at[sl], sem))
        for dsc in descs:
            dsc.wait()
        esl = pl.ds(tbase + g0, GE)
        wdescs = [
            pltpu.async_copy(qb, qg_hbm.at[esl], sem),
            pltpu.async_copy(kb, kg_hbm.at[esl], sem),
            pltpu.async_copy(vb, vg_hbm.at[esl], sem),
            pltpu.async_copy(ab, ag_hbm.at[esl], sem),
            pltpu.async_copy(bb, bg_hbm.at[esl], sem),
        ]
        for dsc in wdescs:
            dsc.wait()
        return carry

    lax.fori_loop(0, gcnt, group, 0)


def _edge_gather(q, k, v, a, b, es3d, ed3d, sn3d, dn3d):
    mesh = plsc.VectorSubcoreMesh(core_axis_name="c", subcore_axis_name="s",
                                  num_cores=NC, num_subcores=NS)
    ew = jax.ShapeDtypeStruct((E, W), jnp.float32)
    return pl.kernel(
        _edge_gather_body,
        out_type=[ew, ew, ew, ew, ew],
        mesh=mesh,
        compiler_params=pltpu.CompilerParams(use_tc_tiling_on_sc=False),
        scratch_types=[
            pltpu.VMEM((K4_CHUNKS * CH,), jnp.int32),
            pltpu.VMEM((K4_CHUNKS * CH,), jnp.int32),
            pltpu.VMEM((K4_CHUNKS * CH,), jnp.int32),
            pltpu.VMEM((K4_CHUNKS * CH,), jnp.int32),
            pltpu.VMEM((GE, W), jnp.float32),
            pltpu.VMEM((GE, W), jnp.float32),
            pltpu.VMEM((GE, W), jnp.float32),
            pltpu.VMEM((GE, W), jnp.float32),
            pltpu.VMEM((GE, W), jnp.float32),
            pltpu.SemaphoreType.DMA,
        ],
    )(q, k, v, a, b, es3d, ed3d, sn3d, dn3d)


def _edge_dense_body(qg_ref, kg_ref, vg_ref, ag_ref, bg_ref, eat_ref,
                     w72_ref, red_ref, wrow_ref, r_ref):
    lane = lax.broadcasted_iota(jnp.int32, (PR, 128), 1) % W
    d = jnp.dot(eat_ref[...], w72_ref[...], preferred_element_type=jnp.float32)
    g = ag_ref[...] + bg_ref[...] + d
    kj = kg_ref[...] + g
    vj = vg_ref[...] + g
    s8 = jnp.dot(qg_ref[...] * kj, red_ref[...],
                 preferred_element_type=jnp.float32)
    ex4 = jnp.exp(s8[:, 0:4] * (1.0 / math.sqrt(float(D))) - _SHIFT)
    exf = jnp.dot(ex4, wrow_ref[2:6, :], preferred_element_type=jnp.float32)
    r_ref[...] = jnp.where(lane < D, exf * vj,
                           jnp.where(lane == D, exf, 0.0))


def _edge_dense(qg, kg, vg, ag, bg, eat4, w72, red, wrow):
    grid = E // BLK
    pflat = pl.BlockSpec((PR, 128), lambda i: (i, 0))
    return pl.pallas_call(
        _edge_dense_body,
        grid=(grid,),
        in_specs=[
            pflat, pflat, pflat, pflat, pflat,
            pl.BlockSpec((PR, 72), lambda i: (i, 0)),
            pl.BlockSpec((72, 128), lambda i: (0, 0)),
            pl.BlockSpec((128, 8), lambda i: (0, 0)),
            pl.BlockSpec((8, 128), lambda i: (0, 0)),
        ],
        out_specs=pflat,
        out_shape=jax.ShapeDtypeStruct((EFR, 128), jnp.float32),
    )(qg, kg, vg, ag, bg, eat4, w72, red, wrow)


def _seg32_body(r_hbm, ed_hbm, zer_hbm, part_hbm, ixd, rb, accum_sh, sem):
    cid = lax.axis_index("c")
    sid = lax.axis_index("s")
    wid = cid * NS + sid
    pltpu.sync_copy(zer_hbm, accum_sh.at[pl.ds(sid * ROWS_PER_TILE,
                                               ROWS_PER_TILE)])
    plsc.subcore_barrier()

    tbase = wid * (K4_CHUNKS * CH)
    pltpu.sync_copy(ed_hbm.at[pl.ds(tbase, K4_CHUNKS * CH)], ixd)

    def group(g, carry):
        g0 = pl.multiple_of(g * GE, 8)
        pltpu.sync_copy(r_hbm.at[pl.ds(tbase + g0, GE)], rb)
        descs = []
        for j in range(K4_G):
            sl = pl.ds(j * CH, CH)
            descs.append(pltpu.async_copy(
                rb.at[sl], accum_sh.at[ixd.at[pl.ds(g0 + j * CH, CH)]],
                sem, add=True))
        for dsc in descs:
            dsc.wait()
        return carry

    lax.fori_loop(0, K4_CHUNKS // K4_G, group, 0)
    plsc.subcore_barrier()
    sl = pl.ds(sid * ROWS_PER_TILE, ROWS_PER_TILE)
    pltpu.sync_copy(accum_sh.at[sl], part_hbm.at[cid].at[sl])


def _seg32(r, ed3d, zer32):
    mesh = plsc.VectorSubcoreMesh(core_axis_name="c", subcore_axis_name="s",
                                  num_cores=NC, num_subcores=NS)
    return pl.kernel(
        _seg32_body,
        out_type=jax.ShapeDtypeStruct((NC, NPAD, W), jnp.float32),
        mesh=mesh,
        compiler_params=pltpu.CompilerParams(use_tc_tiling_on_sc=False),
        scratch_types=[
            pltpu.VMEM((K4_CHUNKS * CH,), jnp.int32),
            pltpu.VMEM((GE, W), jnp.float32),
            pltpu.VMEM_SHARED((NPAD, W), jnp.float32),
            pltpu.SemaphoreType.DMA,
        ],
    )(r, ed3d, zer32)


def _final_body(part_ref, sk_ref, o_ref):
    p = part_ref[0] + part_ref[1]
    den = p[:, D:D + 1]
    num = p[:, 0:D]
    o_ref[...] = num / (den + 1e-16) + sk_ref[...]


def _final(part, sk):
    blk = 2000
    grid = N // blk
    return pl.pallas_call(
        _final_body,
        grid=(grid,),
        in_specs=[
            pl.BlockSpec((NC, blk, W), lambda i: (0, i, 0)),
            pl.BlockSpec((blk, D), lambda i: (i, 0)),
        ],
        out_specs=pl.BlockSpec((blk, D), lambda i: (i, 0)),
        out_shape=jax.ShapeDtypeStruct((N, D), jnp.float32),
    )(part, sk)


@jax.jit
def _run(x, src_n_id, dst_n_id, edge_index, edge_attr, t,
         his_edge_index, his_enc_t, his_h_edge_attr, his_z,
         Wq, bq, Wk, bk, Wv, bv, We, be, Wskip, bskip):
    f32 = jnp.float32
    We_e = We[0:16]
    We_st, We_dt = We[16:21], We[21:26]
    We_sx, We_dx = We[26:154], We[154:282]
    We_h = We[282:285]
    wxp = jnp.concatenate([We_sx, We_dx], axis=1)
    wab = jnp.concatenate([
        jnp.concatenate([-We_st, -We_dt], axis=1),
        jnp.concatenate([We_h, jnp.zeros((3, D), f32)], axis=1),
    ], axis=0)
    wqkvs = jnp.concatenate([
        jnp.concatenate([Wq, Wk, Wv, Wskip], axis=1),
        jnp.concatenate([bq, bk, bv, bskip])[None, :],
        jnp.zeros((3, 4 * D), f32),
    ], axis=0)
    w_t = We_st.sum(0) + We_dt.sum(0)

    wcol = jnp.zeros((W - D,), f32)
    wt32 = jnp.concatenate([w_t, wcol])
    spread4 = jnp.kron(jnp.eye(4, dtype=f32), jnp.ones((1, W), f32))
    wrow = jnp.concatenate([
        jnp.zeros((2, 128), f32),
        spread4,
        jnp.zeros((2, 128), f32),
    ], axis=0)
    wee32 = jnp.concatenate([We_e, jnp.zeros((16, W - D), f32)], axis=1)
    be32 = jnp.concatenate([be, wcol])
    grp18 = jnp.concatenate([wee32, wt32[None, :], be32[None, :]], axis=0)
    w72 = jnp.concatenate([
        jnp.pad(grp18, ((0, 0), (r * W, 128 - W - r * W)))
        for r in range(4)
    ], axis=0)
    redcol = jnp.concatenate([jnp.ones((D,), f32), wcol])[:, None]
    red = jnp.concatenate(
        [jnp.kron(jnp.eye(4, dtype=f32), redcol),
         jnp.zeros((128, 4), f32)], axis=1)

    heh = jnp.concatenate([his_enc_t, his_h_edge_attr], axis=1)
    hz1 = jnp.concatenate([his_z, jnp.ones((N, 1), f32),
                           jnp.zeros((N, 3), f32)], axis=1)

    msrc = jnp.concatenate([edge_index[0], his_edge_index[0]])
    mdst = jnp.concatenate([edge_index[1], his_edge_index[1]])
    es1d, ed1d = edge_index[0], edge_index[1]
    zer40 = jnp.zeros((ROWS_PER_TILE, 40), f32)
    zer32 = jnp.zeros((ROWS_PER_TILE, W), f32)

    xp, ab0, q, k, v, sk = _node_pre(x, heh, hz1, wxp, wab, wqkvs)
    part40 = _seg40(xp, msrc, mdst, zer40)
    a, b = _combine(part40, ab0)
    eat4 = jnp.concatenate([
        edge_attr.reshape(E // 4, 4, 16),
        t.reshape(E // 4, 4, 1),
        jnp.ones((E // 4, 4, 1), f32),
    ], axis=2).reshape(E // 4, 72)
    qg, kg, vg, ag, bg = _edge_gather(q, k, v, a, b,
                                      es1d, ed1d, src_n_id, dst_n_id)
    fl = lambda u: u.reshape(EFR, 128)
    r = _edge_dense(fl(qg), fl(kg), fl(vg), fl(ag), fl(bg), eat4,
                    w72, red, wrow)
    part32 = _seg32(r.reshape(E, W), ed1d, zer32)
    return _final(part32, sk)


def kernel(x, n_id, src_n_id, dst_n_id, edge_index, edge_attr, t, k,
           his_edge_index, his_enc_t, his_h_edge_attr, his_z,
           Wq, bq, Wk, bk, Wv, bv, We, be, Wskip, bskip):
    del n_id, k
    return _run(x, src_n_id, dst_n_id, edge_index, edge_attr, t,
                his_edge_index, his_enc_t, his_h_edge_attr, his_z,
                Wq, bq, Wk, bk, Wv, bv, We, be, Wskip, bskip)


# speedup vs baseline: 1.4696x; 1.4696x over previous
import math

import jax
import jax.numpy as jnp
from jax import lax
from jax.experimental import pallas as pl
from jax.experimental.pallas import tpu as pltpu
from jax.experimental.pallas import tpu_sc as plsc

N = 10000
E = 320000
M = 2 * E
D = 20
W = 32
NC, NS = 2, 16
NW = NC * NS
CH = 80
NPAD = 10240
ROWS_PER_TILE = NPAD // NS

K2_CHUNKS, K2_G = 250, 10
K4_CHUNKS, K4_G = 125, 5
GE = K4_G * CH

BLK = 8000
PR = BLK * W // 128
EFR = E * W // 128

_SHIFT = 15.0


def _node_pre_body(x_ref, heh_ref, hz1_ref, wxp_ref, wab_ref, wqkvs_ref,
                   xp_ref, ab0_ref, q_ref, k_ref, v_ref, sk_ref):
    xp_ref[...] = jnp.dot(x_ref[...], wxp_ref[...],
                          preferred_element_type=jnp.float32)
    ab0_ref[...] = jnp.dot(heh_ref[...], wab_ref[...],
                           preferred_element_type=jnp.float32)
    qkvs = jnp.dot(hz1_ref[...], wqkvs_ref[...],
                   preferred_element_type=jnp.float32)
    zpad = jnp.zeros((qkvs.shape[0], W - D), jnp.float32)
    q_ref[...] = jnp.concatenate([qkvs[:, 0:D], zpad], axis=1)
    k_ref[...] = jnp.concatenate([qkvs[:, D:2 * D], zpad], axis=1)
    v_ref[...] = jnp.concatenate([qkvs[:, 2 * D:3 * D], zpad], axis=1)
    sk_ref[...] = qkvs[:, 3 * D:4 * D]


def _node_pre(x, heh, hz1, wxp, wab, wqkvs):
    blk = 400
    grid = N // blk
    return pl.pallas_call(
        _node_pre_body,
        grid=(grid,),
        in_specs=[
            pl.BlockSpec((blk, 128), lambda i: (i, 0)),
            pl.BlockSpec((blk, 8), lambda i: (i, 0)),
            pl.BlockSpec((blk, 24), lambda i: (i, 0)),
            pl.BlockSpec((128, 40), lambda i: (0, 0)),
            pl.BlockSpec((8, 40), lambda i: (0, 0)),
            pl.BlockSpec((24, 80), lambda i: (0, 0)),
        ],
        out_specs=[
            pl.BlockSpec((blk, 40), lambda i: (i, 0)),
            pl.BlockSpec((blk, 40), lambda i: (i, 0)),
            pl.BlockSpec((blk, W), lambda i: (i, 0)),
            pl.BlockSpec((blk, W), lambda i: (i, 0)),
            pl.BlockSpec((blk, W), lambda i: (i, 0)),
            pl.BlockSpec((blk, D), lambda i: (i, 0)),
        ],
        out_shape=[
            jax.ShapeDtypeStruct((N, 40), jnp.float32),
            jax.ShapeDtypeStruct((N, 40), jnp.float32),
            jax.ShapeDtypeStruct((N, W), jnp.float32),
            jax.ShapeDtypeStruct((N, W), jnp.float32),
            jax.ShapeDtypeStruct((N, W), jnp.float32),
            jax.ShapeDtypeStruct((N, D), jnp.float32),
        ],
    )(x, heh, hz1, wxp, wab, wqkvs)


def _seg40_body(xp_hbm, src_hbm, dst_hbm, zer_hbm, part_hbm,
                idxs_v, idxd_v, rows_v, accum_sh, sem):
    cid = lax.axis_index("c")
    sid = lax.axis_index("s")
    wid = cid * NS + sid
    pltpu.sync_copy(zer_hbm, accum_sh.at[pl.ds(sid * ROWS_PER_TILE,
                                               ROWS_PER_TILE)])
    ebase = wid * (K2_CHUNKS * CH)
    pltpu.sync_copy(src_hbm.at[pl.ds(ebase, K2_CHUNKS * CH)], idxs_v)
    pltpu.sync_copy(dst_hbm.at[pl.ds(ebase, K2_CHUNKS * CH)], idxd_v)
    plsc.subcore_barrier()

    def group(g, carry):
        g0 = pl.multiple_of(g * (K2_G * CH), 8)
        descs = []
        for j in range(K2_G):
            descs.append(pltpu.async_copy(
                xp_hbm.at[idxs_v.at[pl.ds(g0 + j * CH, CH)]],
                rows_v.at[pl.ds(j * CH, CH)], sem))
        for dsc in descs:
            dsc.wait()
        descs = []
        for j in range(K2_G):
            descs.append(pltpu.async_copy(
                rows_v.at[pl.ds(j * CH, CH)],
                accum_sh.at[idxd_v.at[pl.ds(g0 + j * CH, CH)]],
                sem, add=True))
        for dsc in descs:
            dsc.wait()
        return carry

    lax.fori_loop(0, K2_CHUNKS // K2_G, group, 0)
    plsc.subcore_barrier()
    sl = pl.ds(sid * ROWS_PER_TILE, ROWS_PER_TILE)
    pltpu.sync_copy(accum_sh.at[sl], part_hbm.at[cid].at[sl])


def _seg40(xp, src3d, dst3d, zer40):
    mesh = plsc.VectorSubcoreMesh(core_axis_name="c", subcore_axis_name="s",
                                  num_cores=NC, num_subcores=NS)
    return pl.kernel(
        _seg40_body,
        out_type=jax.ShapeDtypeStruct((NC, NPAD, 40), jnp.float32),
        mesh=mesh,
        compiler_params=pltpu.CompilerParams(use_tc_tiling_on_sc=False),
        scratch_types=[
            pltpu.VMEM((K2_CHUNKS * CH,), jnp.int32),
            pltpu.VMEM((K2_CHUNKS * CH,), jnp.int32),
            pltpu.VMEM((K2_G * CH, 40), jnp.float32),
            pltpu.VMEM_SHARED((NPAD, 40), jnp.float32),
            pltpu.SemaphoreType.DMA,
        ],
    )(xp, src3d, dst3d, zer40)


def _combine_body(part_ref, ab0_ref, a_ref, b_ref):
    p = part_ref[0] + part_ref[1] + ab0_ref[...]
    zpad = jnp.zeros((p.shape[0], W - D), jnp.float32)
    a_ref[...] = jnp.concatenate([p[:, 0:D], zpad], axis=1)
    b_ref[...] = jnp.concatenate([p[:, D:2 * D], zpad], axis=1)


def _combine(part, ab0):
    blk = 2000
    grid = N // blk
    return pl.pallas_call(
        _combine_body,
        grid=(grid,),
        in_specs=[
            pl.BlockSpec((NC, blk, 40), lambda i: (0, i, 0)),
            pl.BlockSpec((blk, 40), lambda i: (i, 0)),
        ],
        out_specs=[
            pl.BlockSpec((blk, W), lambda i: (i, 0)),
            pl.BlockSpec((blk, W), lambda i: (i, 0)),
        ],
        out_shape=[
            jax.ShapeDtypeStruct((N, W), jnp.float32),
            jax.ShapeDtypeStruct((N, W), jnp.float32),
        ],
    )(part, ab0)


def _edge_gather_body(q_hbm, k_hbm, v_hbm, a_hbm, b_hbm,
                      es_hbm, ed_hbm, sn_hbm, dn_hbm,
                      qg_hbm, kg_hbm, vg_hbm, ag_hbm, bg_hbm,
                      ixs, ixd, ixsn, ixdn, qb, kb, vb, ab, bb, sem):
    wid = lax.axis_index("c") * NS + lax.axis_index("s")
    gcnt = K4_CHUNKS // K4_G
    tbase = wid * (K4_CHUNKS * CH)
    pltpu.sync_copy(es_hbm.at[pl.ds(tbase, K4_CHUNKS * CH)], ixs)
    pltpu.sync_copy(ed_hbm.at[pl.ds(tbase, K4_CHUNKS * CH)], ixd)
    pltpu.sync_copy(sn_hbm.at[pl.ds(tbase, K4_CHUNKS * CH)], ixsn)
    pltpu.sync_copy(dn_hbm.at[pl.ds(tbase, K4_CHUNKS * CH)], ixdn)

    def group(g, carry):
        g0 = pl.multiple_of(g * GE, 8)
        descs = []
        for j in range(K4_G):
            sl = pl.ds(j * CH, CH)
            il = pl.ds(g0 + j * CH, CH)
            descs.append(pltpu.async_copy(q_hbm.at[ixd.at[il]], qb.at[sl], sem))
            descs.append(pltpu.async_copy(k_hbm.at[ixs.at[il]], kb.at[sl], sem))
            descs.append(pltpu.async_copy(v_hbm.at[ixs.at[il]], vb.at[sl], sem))
            descs.append(pltpu.async_copy(a_hbm.at[ixsn.at[il]], ab.at[sl], sem))
            descs.append(pltpu.async_copy(b_hbm.at[ixdn.at[il]], bb.at[sl], sem))
        for dsc in descs:
            dsc.wait()
        esl = pl.ds(tbase + g0, GE)
        wdescs = [
            pltpu.async_copy(qb, qg_hbm.at[esl], sem),
            pltpu.async_copy(kb, kg_hbm.at[esl], sem),
            pltpu.async_copy(vb, vg_hbm.at[esl], sem),
            pltpu.async_copy(ab, ag_hbm.at[esl], sem),
            pltpu.async_copy(bb, bg_hbm.at[esl], sem),
        ]
        for dsc in wdescs:
            dsc.wait()
        return carry

    lax.fori_loop(0, gcnt, group, 0)


def _edge_gather(q, k, v, a, b, es3d, ed3d, sn3d, dn3d):
    mesh = plsc.VectorSubcoreMesh(core_axis_name="c", subcore_axis_name="s",
                                  num_cores=NC, num_subcores=NS)
    ew = jax.ShapeDtypeStruct((E, W), jnp.float32)
    return pl.kernel(
        _edge_gather_body,
        out_type=[ew, ew, ew, ew, ew],
        mesh=mesh,
        compiler_params=pltpu.CompilerParams(use_tc_tiling_on_sc=False),
        scratch_types=[
            pltpu.VMEM((K4_CHUNKS * CH,), jnp.int32),
            pltpu.VMEM((K4_CHUNKS * CH,), jnp.int32),
            pltpu.VMEM((K4_CHUNKS * CH,), jnp.int32),
            pltpu.VMEM((K4_CHUNKS * CH,), jnp.int32),
            pltpu.VMEM((GE, W), jnp.float32),
            pltpu.VMEM((GE, W), jnp.float32),
            pltpu.VMEM((GE, W), jnp.float32),
            pltpu.VMEM((GE, W), jnp.float32),
            pltpu.VMEM((GE, W), jnp.float32),
            pltpu.SemaphoreType.DMA,
        ],
    )(q, k, v, a, b, es3d, ed3d, sn3d, dn3d)


def _edge_dense_body(qg_ref, kg_ref, vg_ref, ag_ref, bg_ref, eat_ref,
                     w72_ref, red_ref, wrow_ref, r_ref):
    lane = lax.broadcasted_iota(jnp.int32, (PR, 128), 1) % W
    d = jnp.dot(eat_ref[...], w72_ref[...], preferred_element_type=jnp.float32)
    d = d + wrow_ref[1:2, :]
    g = ag_ref[...] + bg_ref[...] + d
    kj = kg_ref[...] + g
    vj = vg_ref[...] + g
    s8 = jnp.dot(qg_ref[...] * kj, red_ref[...],
                 preferred_element_type=jnp.float32)
    ex4 = jnp.exp(s8[:, 0:4] * (1.0 / math.sqrt(float(D))) - _SHIFT)
    exf = jnp.dot(ex4, wrow_ref[2:6, :], preferred_element_type=jnp.float32)
    r_ref[...] = jnp.where(lane < D, exf * vj,
                           jnp.where(lane == D, exf, 0.0))


def _edge_dense(qg, kg, vg, ag, bg, eat4, w72, red, wrow):
    grid = E // BLK
    pflat = pl.BlockSpec((PR, 128), lambda i: (i, 0))
    return pl.pallas_call(
        _edge_dense_body,
        grid=(grid,),
        in_specs=[
            pflat, pflat, pflat, pflat, pflat,
            pl.BlockSpec((PR, 72), lambda i: (i, 0)),
            pl.BlockSpec((72, 128), lambda i: (0, 0)),
            pl.BlockSpec((128, 8), lambda i: (0, 0)),
            pl.BlockSpec((8, 128), lambda i: (0, 0)),
        ],
        out_specs=pflat,
        out_shape=jax.ShapeDtypeStruct((EFR, 128), jnp.float32),
    )(qg, kg, vg, ag, bg, eat4, w72, red, wrow)


def _seg32_body(r_hbm, ed_hbm, zer_hbm, part_hbm, ixd, rb, accum_sh, sem):
    cid = lax.axis_index("c")
    sid = lax.axis_index("s")
    wid = cid * NS + sid
    pltpu.sync_copy(zer_hbm, accum_sh.at[pl.ds(sid * ROWS_PER_TILE,
                                               ROWS_PER_TILE)])
    plsc.subcore_barrier()

    tbase = wid * (K4_CHUNKS * CH)
    pltpu.sync_copy(ed_hbm.at[pl.ds(tbase, K4_CHUNKS * CH)], ixd)

    def group(g, carry):
        g0 = pl.multiple_of(g * GE, 8)
        pltpu.sync_copy(r_hbm.at[pl.ds(tbase + g0, GE)], rb)
        descs = []
        for j in range(K4_G):
            sl = pl.ds(j * CH, CH)
            descs.append(pltpu.async_copy(
                rb.at[sl], accum_sh.at[ixd.at[pl.ds(g0 + j * CH, CH)]],
                sem, add=True))
        for dsc in descs:
            dsc.wait()
        return carry

    lax.fori_loop(0, K4_CHUNKS // K4_G, group, 0)
    plsc.subcore_barrier()
    sl = pl.ds(sid * ROWS_PER_TILE, ROWS_PER_TILE)
    pltpu.sync_copy(accum_sh.at[sl], part_hbm.at[cid].at[sl])


def _seg32(r, ed3d, zer32):
    mesh = plsc.VectorSubcoreMesh(core_axis_name="c", subcore_axis_name="s",
                                  num_cores=NC, num_subcores=NS)
    return pl.kernel(
        _seg32_body,
        out_type=jax.ShapeDtypeStruct((NC, NPAD, W), jnp.float32),
        mesh=mesh,
        compiler_params=pltpu.CompilerParams(use_tc_tiling_on_sc=False),
        scratch_types=[
            pltpu.VMEM((K4_CHUNKS * CH,), jnp.int32),
            pltpu.VMEM((GE, W), jnp.float32),
            pltpu.VMEM_SHARED((NPAD, W), jnp.float32),
            pltpu.SemaphoreType.DMA,
        ],
    )(r, ed3d, zer32)


def _final_body(part_ref, sk_ref, o_ref):
    p = part_ref[0] + part_ref[1]
    den = p[:, D:D + 1]
    num = p[:, 0:D]
    o_ref[...] = num / (den + 1e-16) + sk_ref[...]


def _final(part, sk):
    blk = 2000
    grid = N // blk
    return pl.pallas_call(
        _final_body,
        grid=(grid,),
        in_specs=[
            pl.BlockSpec((NC, blk, W), lambda i: (0, i, 0)),
            pl.BlockSpec((blk, D), lambda i: (i, 0)),
        ],
        out_specs=pl.BlockSpec((blk, D), lambda i: (i, 0)),
        out_shape=jax.ShapeDtypeStruct((N, D), jnp.float32),
    )(part, sk)


@jax.jit
def _run(x, src_n_id, dst_n_id, edge_index, edge_attr, t,
         his_edge_index, his_enc_t, his_h_edge_attr, his_z,
         Wq, bq, Wk, bk, Wv, bv, We, be, Wskip, bskip):
    f32 = jnp.float32
    We_e = We[0:16]
    We_st, We_dt = We[16:21], We[21:26]
    We_sx, We_dx = We[26:154], We[154:282]
    We_h = We[282:285]
    wxp = jnp.concatenate([We_sx, We_dx], axis=1)
    wab = jnp.concatenate([
        jnp.concatenate([-We_st, -We_dt], axis=1),
        jnp.concatenate([We_h, jnp.zeros((3, D), f32)], axis=1),
    ], axis=0)
    wqkvs = jnp.concatenate([
        jnp.concatenate([Wq, Wk, Wv, Wskip], axis=1),
        jnp.concatenate([bq, bk, bv, bskip])[None, :],
        jnp.zeros((3, 4 * D), f32),
    ], axis=0)
    w_t = We_st.sum(0) + We_dt.sum(0)

    wcol = jnp.zeros((W - D,), f32)
    wt32 = jnp.concatenate([w_t, wcol])
    be32 = jnp.concatenate([be, wcol])
    spread4 = jnp.kron(jnp.eye(4, dtype=f32), jnp.ones((1, W), f32))
    wrow = jnp.concatenate([
        jnp.tile(wt32, 4)[None, :],
        jnp.tile(be32, 4)[None, :],
        spread4,
        jnp.zeros((2, 128), f32),
    ], axis=0)
    wee32 = jnp.concatenate([We_e, jnp.zeros((16, W - D), f32)], axis=1)
    w64 = jnp.concatenate([
        jnp.pad(wee32, ((0, 0), (r * W, 128 - W - r * W)))
        for r in range(4)
    ], axis=0)
    wt_rows = jnp.concatenate([
        jnp.pad(wt32[None, :], ((0, 0), (r * W, 128 - W - r * W)))
        for r in range(4)
    ], axis=0)
    w72 = jnp.concatenate([w64, wt_rows, jnp.zeros((4, 128), f32)],
                          axis=0)
    redcol = jnp.concatenate([jnp.ones((D,), f32), wcol])[:, None]
    red = jnp.concatenate(
        [jnp.kron(jnp.eye(4, dtype=f32), redcol),
         jnp.zeros((128, 4), f32)], axis=1)

    heh = jnp.concatenate([his_enc_t, his_h_edge_attr], axis=1)
    hz1 = jnp.concatenate([his_z, jnp.ones((N, 1), f32),
                           jnp.zeros((N, 3), f32)], axis=1)

    msrc = jnp.concatenate([edge_index[0], his_edge_index[0]])
    mdst = jnp.concatenate([edge_index[1], his_edge_index[1]])
    es1d, ed1d = edge_index[0], edge_index[1]
    zer40 = jnp.zeros((ROWS_PER_TILE, 40), f32)
    zer32 = jnp.zeros((ROWS_PER_TILE, W), f32)

    xp, ab0, q, k, v, sk = _node_pre(x, heh, hz1, wxp, wab, wqkvs)
    part40 = _seg40(xp, msrc, mdst, zer40)
    a, b = _combine(part40, ab0)
    eat4 = jnp.concatenate([
        edge_attr.reshape(E // 4, 64),
        t.reshape(E // 4, 4),
        jnp.zeros((E // 4, 4), f32),
    ], axis=1)
    qg, kg, vg, ag, bg = _edge_gather(q, k, v, a, b,
                                      es1d, ed1d, src_n_id, dst_n_id)
    fl = lambda u: u.reshape(EFR, 128)
    r = _edge_dense(fl(qg), fl(kg), fl(vg), fl(ag), fl(bg), eat4,
                    w72, red, wrow)
    part32 = _seg32(r.reshape(E, W), ed1d, zer32)
    return _final(part32, sk)


def kernel(x, n_id, src_n_id, dst_n_id, edge_index, edge_attr, t, k,
           his_edge_index, his_enc_t, his_h_edge_attr, his_z,
           Wq, bq, Wk, bk, Wv, bv, We, be, Wskip, bskip):
    del n_id, k
    return _run(x, src_n_id, dst_n_id, edge_index, edge_attr, t,
                his_edge_index, his_enc_t, his_h_edge_attr, his_z,
                Wq, bq, Wk, bk, Wv, bv, We, be, Wskip, bskip)
